# baseline (speedup 1.0000x reference)
import functools

import jax
import jax.numpy as jnp
from jax import lax
from jax.experimental import pallas as pl
from jax.experimental.pallas import tpu as pltpu
from jax.experimental.pallas import tpu_sc as plsc

EMBED = 64
NC, NS = 2, 16
NW = NC * NS
TOTAL = 4096 * 200
PER_W = TOTAL // NW
CHUNK = 512
KSUB = CHUNK // 128
NCHUNK = PER_W // CHUNK


@functools.partial(
    pl.kernel,
    mesh=plsc.VectorSubcoreMesh(core_axis_name="c", subcore_axis_name="s"),
    out_type=jax.ShapeDtypeStruct((TOTAL, EMBED), jnp.float32),
    scratch_types=[
        pltpu.VMEM((CHUNK,), jnp.int32),
        pltpu.VMEM((CHUNK, EMBED), jnp.float32),
        pltpu.SemaphoreType.DMA,
    ],
    compiler_params=pltpu.CompilerParams(use_tc_tiling_on_sc=False),
)
def _sc_gather(table_hbm, idx_hbm, out_hbm, idx_v, rows_v, sem):
    wid = lax.axis_index("s") * NC + lax.axis_index("c")
    base = wid * PER_W

    def chunk_body(c, carry):
        off = base + c * CHUNK
        pltpu.sync_copy(idx_hbm.at[pl.ds(off, CHUNK)], idx_v)
        cps = [
            pltpu.async_copy(
                table_hbm.at[idx_v.at[pl.ds(j * 128, 128)]],
                rows_v.at[pl.ds(j * 128, 128), :],
                sem,
            )
            for j in range(KSUB)
        ]
        for cp in cps:
            cp.wait()
        pltpu.sync_copy(rows_v, out_hbm.at[pl.ds(off, CHUNK), :])
        return carry

    lax.fori_loop(0, NCHUNK, chunk_body, 0)


def kernel(tokens, wte_weight):
    b, s = tokens.shape
    idx1d = tokens.astype(jnp.int32).reshape(TOTAL)
    out = _sc_gather(wte_weight, idx1d)
    return out.reshape(b, s, EMBED)


# speedup vs baseline: 1.0434x; 1.0434x over previous
import functools

import jax
import jax.numpy as jnp
from jax import lax
from jax.experimental import pallas as pl
from jax.experimental.pallas import tpu as pltpu
from jax.experimental.pallas import tpu_sc as plsc

EMBED = 64
NC, NS = 2, 16
NW = NC * NS
TOTAL = 4096 * 200
PER_W = TOTAL // NW
CHUNK = 512
KSUB = CHUNK // 128
NCHUNK = PER_W // CHUNK
NBUF = 3


@functools.partial(
    pl.kernel,
    mesh=plsc.VectorSubcoreMesh(core_axis_name="c", subcore_axis_name="s"),
    out_type=jax.ShapeDtypeStruct((TOTAL, EMBED), jnp.float32),
    scratch_types=[
        pltpu.VMEM((PER_W,), jnp.int32),
        pltpu.VMEM((NBUF, CHUNK, EMBED), jnp.float32),
        pltpu.SemaphoreType.DMA,
        pltpu.SemaphoreType.DMA,
        pltpu.SemaphoreType.DMA,
        pltpu.SemaphoreType.DMA,
        pltpu.SemaphoreType.DMA,
        pltpu.SemaphoreType.DMA,
    ],
    compiler_params=pltpu.CompilerParams(use_tc_tiling_on_sc=False),
)
def _sc_gather(table_hbm, idx_hbm, out_hbm, idx_v, rows_v, g0, g1, g2,
               o0, o1, o2):
    gsem = (g0, g1, g2)
    osem = (o0, o1, o2)
    wid = lax.axis_index("s") * NC + lax.axis_index("c")
    base = wid * PER_W
    pltpu.sync_copy(idx_hbm.at[pl.ds(base, PER_W)], idx_v)

    def fire_g(k, b):
        for j in range(KSUB):
            pltpu.async_copy(
                table_hbm.at[idx_v.at[pl.ds(k * CHUNK + j * 128, 128)]],
                rows_v.at[b, pl.ds(j * 128, 128), :],
                gsem[b],
            )

    def wait_g(b):
        for j in range(KSUB):
            pltpu.make_async_copy(
                out_hbm.at[pl.ds(0, 128), :],
                rows_v.at[b, pl.ds(j * 128, 128), :],
                gsem[b],
            ).wait()

    def fire_o(k, b):
        pltpu.async_copy(
            rows_v.at[b],
            out_hbm.at[pl.ds(base + k * CHUNK, CHUNK), :],
            osem[b],
        )

    def wait_o(b):
        pltpu.make_async_copy(
            rows_v.at[b],
            out_hbm.at[pl.ds(base, CHUNK), :],
            osem[b],
        ).wait()

    def step(k, b):
        wait_g(b)
        fire_o(k, b)
        bn = (b + 2) % NBUF
        wait_o(bn)
        fire_g(k + 2, bn)

    fire_g(0, 0)
    fire_g(1, 1)
    wait_g(0)
    fire_o(0, 0)
    fire_g(2, 2)
    first, last = 1, NCHUNK - 3
    ngrp = (last - first + 1) // 3

    def grp(g, carry):
        k = first + g * 3
        step(k, 1)
        step(k + 1, 2)
        step(k + 2, 0)
        return carry

    lax.fori_loop(0, ngrp, grp, 0)
    for k in range(first + ngrp * 3, last + 1):
        step(k, k % NBUF)
    for k in (NCHUNK - 2, NCHUNK - 1):
        wait_g(k % NBUF)
        fire_o(k, k % NBUF)
    for b in range(NBUF):
        wait_o(b)


def kernel(tokens, wte_weight):
    b, s = tokens.shape
    idx1d = tokens.astype(jnp.int32).reshape(TOTAL)
    out = _sc_gather(wte_weight, idx1d)
    return out.reshape(b, s, EMBED)
